# baseline (speedup 1.0000x reference)
import functools

import jax
import jax.numpy as jnp
from jax import lax
from jax.experimental import pallas as pl
from jax.experimental.pallas import tpu as pltpu
from jax.experimental.pallas import tpu_sc as plsc

_B, _S, _V = 1024, 50, 1000
_ROWS = _B * _S
_L = 16
_NBUF = 4
_TILE = _L * _V


def _onehot_body(nw, rpw, idx_hbm, zeros_hbm, out_hbm, idx_v, buf, *sems):
    nsteps = rpw // _L // _NBUF
    wid = lax.axis_index("s") * (nw // 16) + lax.axis_index("c")
    base = wid * rpw

    pltpu.sync_copy(idx_hbm.at[pl.ds(base, rpw)], idx_v)
    pltpu.sync_copy(zeros_hbm, buf)

    lane_off = lax.iota(jnp.int32, 16) * _V
    ones = jnp.ones((_L,), jnp.float32)
    zeros = jnp.zeros((_L,), jnp.float32)

    def fire(step, b):
        rowbase = pl.multiple_of((step * _NBUF + b) * _L, _L)
        idxvec = idx_v[pl.ds(rowbase, _L)]
        plsc.store_scatter(buf, [lane_off + b * _TILE + idxvec], ones)
        pltpu.async_copy(
            buf.at[pl.ds(b * _TILE, _TILE)],
            out_hbm.at[pl.ds((base + rowbase) * _V, _TILE)],
            sems[b],
        )

    def wait(b):
        pltpu.make_async_copy(
            buf.at[pl.ds(b * _TILE, _TILE)],
            out_hbm.at[pl.ds(base * _V, _TILE)],
            sems[b],
        ).wait()

    def reset(step, b):
        prevbase = pl.multiple_of((step * _NBUF + b - _NBUF) * _L, _L)
        prev = idx_v[pl.ds(prevbase, _L)]
        plsc.store_scatter(buf, [lane_off + b * _TILE + prev], zeros)

    for b in range(_NBUF):
        fire(0, b)

    def body(step, carry):
        for b in range(_NBUF):
            wait(b)
            reset(step, b)
            fire(step, b)
        return carry

    lax.fori_loop(1, nsteps, body, 0, unroll=False)

    for b in range(_NBUF):
        wait(b)


def _onehot_sc(idx, zeros):
    info = plsc.get_sparse_core_info()
    nw = info.num_cores * info.num_subcores
    rpw = _ROWS // nw
    mesh = plsc.VectorSubcoreMesh(core_axis_name="c", subcore_axis_name="s")
    k = functools.partial(
        pl.kernel,
        mesh=mesh,
        compiler_params=pltpu.CompilerParams(needs_layout_passes=False),
        out_type=jax.ShapeDtypeStruct((_ROWS * _V,), jnp.float32),
        scratch_types=[
            pltpu.VMEM((rpw,), jnp.int32),
            pltpu.VMEM((_NBUF * _TILE,), jnp.float32),
        ] + [pltpu.SemaphoreType.DMA] * _NBUF,
    )(functools.partial(_onehot_body, nw, rpw))
    return k(idx, zeros)


def kernel(inputs):
    idx = jnp.ravel(inputs).astype(jnp.int32)
    zeros = jnp.zeros((_NBUF * _TILE,), jnp.float32)
    out = _onehot_sc(idx, zeros)
    return out.reshape(_B, _S, _V)


# speedup vs baseline: 1.8832x; 1.8832x over previous
import functools

import jax
import jax.numpy as jnp
from jax import lax
from jax.experimental import pallas as pl
from jax.experimental.pallas import tpu as pltpu
from jax.experimental.pallas import tpu_sc as plsc

_B, _S, _V = 1024, 50, 1000
_L = 16
_NBUF = 2


def _onehot_body(nw, bpw, idx_hbm, zeros_hbm, out_hbm, idx_v, buf, *sems):
    nsteps = bpw // _NBUF
    wid = lax.axis_index("s") * (nw // 16) + lax.axis_index("c")
    base = wid * bpw

    pltpu.sync_copy(idx_hbm.at[pl.ds(base * _S, bpw * _S)], idx_v)
    for b in range(_NBUF):
        pltpu.sync_copy(zeros_hbm, buf.at[b])

    lanes = lax.iota(jnp.int32, 16)
    ones = jnp.ones((_L,), jnp.float32)
    zeros = jnp.zeros((_L,), jnp.float32)
    groups = ((0, None), (16, None), (32, None),
              (40, (lanes >= 8) & (lanes < 10)))

    def scatter(g, b, val):
        for off, mask in groups:
            rows = lanes + off
            idxvec = idx_v[pl.ds(g * _S + off, _L)]
            plsc.store_scatter(buf.at[b], [rows, idxvec], val, mask=mask)

    def fire(g, b):
        scatter(g, b, ones)
        pltpu.async_copy(buf.at[b], out_hbm.at[base + g], sems[b])

    def wait(b):
        pltpu.make_async_copy(buf.at[b], out_hbm.at[base], sems[b]).wait()

    for b in range(_NBUF):
        fire(b, b)

    def body(step, carry):
        for b in range(_NBUF):
            g = step * _NBUF + b
            wait(b)
            scatter(g - _NBUF, b, zeros)
            fire(g, b)
        return carry

    lax.fori_loop(1, nsteps, body, 0, unroll=False)

    for b in range(_NBUF):
        wait(b)


def _onehot_sc(idx, zeros):
    info = plsc.get_sparse_core_info()
    nw = info.num_cores * info.num_subcores
    bpw = _B // nw
    mesh = plsc.VectorSubcoreMesh(core_axis_name="c", subcore_axis_name="s")
    k = functools.partial(
        pl.kernel,
        mesh=mesh,
        compiler_params=pltpu.CompilerParams(needs_layout_passes=False),
        out_type=jax.ShapeDtypeStruct((_B, _S, _V), jnp.float32),
        scratch_types=[
            pltpu.VMEM((bpw * _S,), jnp.int32),
            pltpu.VMEM((_NBUF, _S, _V), jnp.float32),
        ] + [pltpu.SemaphoreType.DMA] * _NBUF,
    )(functools.partial(_onehot_body, nw, bpw))
    return k(idx, zeros)


def kernel(inputs):
    idx = jnp.ravel(inputs).astype(jnp.int32)
    zeros = jnp.zeros((_S, _V), jnp.float32)
    return _onehot_sc(idx, zeros)
